# baseline (speedup 1.0000x reference)
import functools

import jax
import jax.numpy as jnp
from jax import lax
from jax.experimental import pallas as pl
from jax.experimental.pallas import tpu as pltpu
from jax.experimental.pallas import tpu_sc as plsc

_L = 16


@functools.lru_cache(maxsize=None)
def _build(h, w, n):
    info = plsc.get_sparse_core_info()
    nc, ns = info.num_cores, info.num_subcores
    nw = nc * ns
    per_w = n // nw
    c = min(4096, per_w)
    nchunk = per_w // c
    hw = h * w
    mesh = plsc.VectorSubcoreMesh(core_axis_name="c", subcore_axis_name="s")

    @functools.partial(
        pl.kernel,
        mesh=mesh,
        out_type=jax.ShapeDtypeStruct((nw, _L), jnp.float32),
        scratch_types=[
            pltpu.VMEM((c,), jnp.float32),
            pltpu.VMEM((c,), jnp.float32),
            pltpu.VMEM((c,), jnp.int32),
            pltpu.VMEM((c,), jnp.int32),
            pltpu.VMEM((c,), jnp.float32),
            pltpu.VMEM((c,), jnp.float32),
            pltpu.VMEM((_L,), jnp.float32),
            pltpu.SemaphoreType.DMA,
            pltpu.SemaphoreType.DMA,
        ],
    )
    def bouncer(tab, xs, ys, out, xv, yv, iyv, ixv, tyv, txv, accv, sem1, sem2):
        wid = lax.axis_index("s") * nc + lax.axis_index("c")
        base = wid * per_w

        def chunk(k, acc):
            off = base + k * c
            pltpu.sync_copy(xs.at[pl.ds(off, c)], xv)
            pltpu.sync_copy(ys.at[pl.ds(off, c)], yv)

            def mkidx(i, carry):
                s = pl.ds(pl.multiple_of(i * _L, _L), _L)
                xi = jnp.clip(xv[s], 0.0, w - 1).astype(jnp.int32)
                yi = jnp.clip(yv[s], 0.0, h - 1).astype(jnp.int32)
                lin = yi * w + xi
                iyv[s] = lin
                ixv[s] = lin + hw
                return carry

            lax.fori_loop(0, c // _L, mkidx, 0)
            cp1 = pltpu.async_copy(tab.at[iyv], tyv, sem1)
            cp2 = pltpu.async_copy(tab.at[ixv], txv, sem2)
            cp1.wait()
            cp2.wait()

            def accum(i, a):
                s = pl.ds(pl.multiple_of(i * _L, _L), _L)
                dx = xv[s] - txv[s]
                dy = yv[s] - tyv[s]
                return a + dx * dx + dy * dy

            return lax.fori_loop(0, c // _L, accum, acc)

        acc = lax.fori_loop(0, nchunk, chunk, jnp.zeros((_L,), jnp.float32))
        accv[...] = acc
        pltpu.sync_copy(accv, out.at[wid])

    return bouncer


def kernel(dtxy, x, y):
    h, w = dtxy.shape[1], dtxy.shape[2]
    n = x.shape[0]
    tab = dtxy.reshape(-1)
    part = _build(h, w, n)(tab, x, y)
    return jnp.sum(part) / (2.0 * n)


# speedup vs baseline: 1.1072x; 1.1072x over previous
import functools

import jax
import jax.numpy as jnp
from jax import lax
from jax.experimental import pallas as pl
from jax.experimental.pallas import tpu as pltpu
from jax.experimental.pallas import tpu_sc as plsc

_L = 16


@functools.lru_cache(maxsize=None)
def _build(h, w, n):
    info = plsc.get_sparse_core_info()
    nc, ns = info.num_cores, info.num_subcores
    nw = nc * ns
    per_w = n // nw
    c = min(8192, per_w)
    nchunk = per_w // c
    assert nchunk >= 4 and nchunk % 2 == 0
    nvec = c // _L
    mesh = plsc.VectorSubcoreMesh(core_axis_name="c", subcore_axis_name="s")

    @functools.partial(
        pl.kernel,
        mesh=mesh,
        out_type=jax.ShapeDtypeStruct((nw, _L), jnp.float32),
        scratch_types=[
            pltpu.VMEM((c,), jnp.float32),
            pltpu.VMEM((c,), jnp.float32),
            pltpu.VMEM((c,), jnp.float32),
            pltpu.VMEM((c,), jnp.float32),
            pltpu.VMEM((c,), jnp.int32),
            pltpu.VMEM((c,), jnp.int32),
            pltpu.VMEM((c,), jnp.float32),
            pltpu.VMEM((c,), jnp.float32),
            pltpu.VMEM((c,), jnp.float32),
            pltpu.VMEM((c,), jnp.float32),
            pltpu.VMEM((_L,), jnp.float32),
            pltpu.SemaphoreType.DMA((2,)),
            pltpu.SemaphoreType.DMA((2,)),
            pltpu.SemaphoreType.DMA((2,)),
            pltpu.SemaphoreType.DMA((2,)),
        ],
    )
    def bouncer(tab0, tab1, xs, ys, out, xv0, xv1, yv0, yv1, iv0, iv1,
                tyv0, tyv1, txv0, txv1, accv, semx, semy, semt, semu):
        wid = lax.axis_index("s") * nc + lax.axis_index("c")
        base = wid * per_w
        xvs, yvs = (xv0, xv1), (yv0, yv1)
        ivs, tyvs, txvs = (iv0, iv1), (tyv0, tyv1), (txv0, txv1)

        def xy_copies(k, b):
            off = base + k * c
            return (
                pltpu.make_async_copy(xs.at[pl.ds(off, c)], xvs[b], semx.at[b]),
                pltpu.make_async_copy(ys.at[pl.ds(off, c)], yvs[b], semy.at[b]),
            )

        def gather_copies(b):
            return (
                pltpu.make_async_copy(tab0.at[ivs[b]], tyvs[b], semt.at[b]),
                pltpu.make_async_copy(tab1.at[ivs[b]], txvs[b], semu.at[b]),
            )

        def mkidx(b):
            xv, yv, iv = xvs[b], yvs[b], ivs[b]

            def body(i, carry):
                s = pl.ds(pl.multiple_of(i * _L, _L), _L)
                xi = jnp.clip(xv[s], 0.0, w - 1).astype(jnp.int32)
                yi = jnp.clip(yv[s], 0.0, h - 1).astype(jnp.int32)
                iv[s] = yi * w + xi
                return carry

            lax.fori_loop(0, nvec, body, 0, unroll=4)

        def accum(b, acc):
            xv, yv, tyv, txv = xvs[b], yvs[b], tyvs[b], txvs[b]

            def body(i, a):
                s = pl.ds(pl.multiple_of(i * _L, _L), _L)
                dx = xv[s] - txv[s]
                dy = yv[s] - tyv[s]
                return a + dx * dx + dy * dy

            return lax.fori_loop(0, nvec, body, acc, unroll=4)

        def step(k, acc, b, nb, has_next, has_next2):
            if has_next:
                for cp in xy_copies(k + 1, nb):
                    cp.wait()
                mkidx(nb)
                for cp in gather_copies(nb):
                    cp.start()
            for cp in gather_copies(b):
                cp.wait()
            acc = accum(b, acc)
            if has_next2:
                for cp in xy_copies(k + 2, b):
                    cp.start()
            return acc

        for cp in xy_copies(0, 0):
            cp.start()
        for cp in xy_copies(1, 1):
            cp.start()
        for cp in xy_copies(0, 0):
            cp.wait()
        mkidx(0)
        for cp in gather_copies(0):
            cp.start()

        def loop_body(k0, acc):
            k = k0 * 2
            acc = step(k, acc, 0, 1, True, True)
            return step(k + 1, acc, 1, 0, True, True)

        acc = lax.fori_loop(0, (nchunk - 2) // 2, loop_body,
                            jnp.zeros((_L,), jnp.float32))
        acc = step(nchunk - 2, acc, 0, 1, True, False)
        acc = step(nchunk - 1, acc, 1, 0, False, False)

        accv[...] = acc
        pltpu.sync_copy(accv, out.at[wid])

    return bouncer


def kernel(dtxy, x, y):
    h, w = dtxy.shape[1], dtxy.shape[2]
    n = x.shape[0]
    part = _build(h, w, n)(dtxy[0].reshape(-1), dtxy[1].reshape(-1), x, y)
    return jnp.sum(part) / (2.0 * n)
